# baseline (speedup 1.0000x reference)
import jax
import jax.numpy as jnp
from jax.experimental import pallas as pl
from jax.experimental.pallas import tpu as pltpu

_SHIFT = 8192
_LANES = 128


def _copy_body(in_ref, out_ref):
    out_ref[...] = in_ref[:, : out_ref.shape[1], :]


def kernel(wav):
    s, b, c, length = wav.shape
    out_len = length - _SHIFT
    rows = s * b * c
    in_sub = length // _LANES
    out_sub = out_len // _LANES
    x = wav.reshape(rows, in_sub, _LANES)

    out = pl.pallas_call(
        _copy_body,
        grid=(rows,),
        in_specs=[pl.BlockSpec((1, in_sub, _LANES), lambda i: (i, 0, 0))],
        out_specs=pl.BlockSpec((1, out_sub, _LANES), lambda i: (i, 0, 0)),
        out_shape=jax.ShapeDtypeStruct((rows, out_sub, _LANES), wav.dtype),
        compiler_params=pltpu.CompilerParams(
            dimension_semantics=("parallel",),
        ),
    )(x)
    return out.reshape(s, b, c, out_len)


# speedup vs baseline: 1.0980x; 1.0980x over previous
import jax
import jax.numpy as jnp
from jax.experimental import pallas as pl
from jax.experimental.pallas import tpu as pltpu

_SHIFT = 8192
_LANES = 128
_AHEAD = 8
_LAG = 8
_DEPTH = _AHEAD + _LAG


def _copy_body(in_ref, out_ref, vmem, rsem, wsem):
    rows, out_sub, _ = out_ref.shape

    def start_read(j):
        pltpu.make_async_copy(
            in_ref.at[j], vmem.at[j % _DEPTH], rsem.at[j % _DEPTH]
        ).start()

    def wait_read(j):
        pltpu.make_async_copy(
            in_ref.at[j], vmem.at[j % _DEPTH], rsem.at[j % _DEPTH]
        ).wait()

    def _write_copy(j):
        return pltpu.make_async_copy(
            vmem.at[j % _DEPTH, pl.ds(0, out_sub), slice(None)],
            out_ref.at[j],
            wsem.at[j % _DEPTH],
        )

    for j in range(min(_AHEAD, rows)):
        start_read(j)
    for i in range(rows):
        j = i + _AHEAD
        if j < rows:
            if j >= _DEPTH:
                _write_copy(j - _DEPTH).wait()
            start_read(j)
        wait_read(i)
        _write_copy(i).start()
    for i in range(max(0, rows - _DEPTH), rows):
        _write_copy(i).wait()


def kernel(wav):
    s, b, c, length = wav.shape
    out_len = length - _SHIFT
    rows = s * b * c
    in_sub = length // _LANES
    out_sub = out_len // _LANES
    x = wav.reshape(rows, in_sub, _LANES)

    out = pl.pallas_call(
        _copy_body,
        in_specs=[pl.BlockSpec(memory_space=pl.ANY)],
        out_specs=pl.BlockSpec(memory_space=pl.ANY),
        out_shape=jax.ShapeDtypeStruct((rows, out_sub, _LANES), wav.dtype),
        scratch_shapes=[
            pltpu.VMEM((_DEPTH, in_sub, _LANES), wav.dtype),
            pltpu.SemaphoreType.DMA((_DEPTH,)),
            pltpu.SemaphoreType.DMA((_DEPTH,)),
        ],
    )(x)
    return out.reshape(s, b, c, out_len)


# speedup vs baseline: 1.3300x; 1.2113x over previous
import functools

import jax
import jax.numpy as jnp
from jax import lax
from jax.experimental import pallas as pl
from jax.experimental.pallas import tpu as pltpu
from jax.experimental.pallas import tpu_sc as plsc

_SHIFT = 8192

_NC = 2
_NS = 16
_NW = _NC * _NS

_NBUF = 5
_AHEAD = 3


def _make_sc_copy(rows, in_len, out_len, dtype):
    rows_per_w = rows // _NW
    nch = 8
    ch = out_len // nch
    assert ch * nch == out_len and ch % 8 == 0
    ntask = rows_per_w * nch

    mesh = plsc.VectorSubcoreMesh(core_axis_name="c", subcore_axis_name="s")

    @functools.partial(
        pl.kernel,
        out_type=jax.ShapeDtypeStruct((rows * out_len,), dtype),
        mesh=mesh,
        scratch_types=[pltpu.VMEM((ch,), dtype)] * _NBUF + [
            pltpu.SemaphoreType.DMA((_NBUF,)),
            pltpu.SemaphoreType.DMA((_NBUF,)),
        ],
    )
    def sc_copy(in_hbm, out_hbm, *rest):
        bufs, (rsem, wsem) = rest[:_NBUF], rest[_NBUF:]
        wid = lax.axis_index("s") * _NC + lax.axis_index("c")
        base_row = wid * rows_per_w

        def read_copy(t):
            row = base_row + t // nch
            off = pl.multiple_of(row * in_len + (t % nch) * ch, 8)
            slot = t % _NBUF
            return pltpu.make_async_copy(
                in_hbm.at[pl.ds(off, ch)], bufs[slot], rsem.at[slot]
            )

        def write_copy(t):
            row = base_row + t // nch
            off = pl.multiple_of(row * out_len + (t % nch) * ch, 8)
            slot = t % _NBUF
            return pltpu.make_async_copy(
                bufs[slot], out_hbm.at[pl.ds(off, ch)], wsem.at[slot]
            )

        for t in range(min(_AHEAD, ntask)):
            read_copy(t).start()
        for t in range(ntask):
            nt = t + _AHEAD
            if nt < ntask:
                if nt >= _NBUF:
                    write_copy(nt - _NBUF).wait()
                read_copy(nt).start()
            read_copy(t).wait()
            write_copy(t).start()
        for t in range(max(0, ntask - _NBUF), ntask):
            write_copy(t).wait()

    return sc_copy


def kernel(wav):
    s, b, c, length = wav.shape
    out_len = length - _SHIFT
    rows = s * b * c
    x = wav.reshape(rows * length)
    out = _make_sc_copy(rows, length, out_len, wav.dtype)(x)
    return out.reshape(s, b, c, out_len)


# speedup vs baseline: 1.3372x; 1.0054x over previous
import functools

import jax
import jax.numpy as jnp
from jax import lax
from jax.experimental import pallas as pl
from jax.experimental.pallas import tpu as pltpu
from jax.experimental.pallas import tpu_sc as plsc

_SHIFT = 8192

_NC = 2
_NS = 16
_NW = _NC * _NS

_NBUF = 3
_AHEAD = 2


def _make_sc_copy(rows, in_len, out_len, dtype):
    rows_per_w = rows // _NW
    nch = 4
    ch = out_len // nch
    assert ch * nch == out_len and ch % 8 == 0
    ntask = rows_per_w * nch

    mesh = plsc.VectorSubcoreMesh(core_axis_name="c", subcore_axis_name="s")

    @functools.partial(
        pl.kernel,
        out_type=jax.ShapeDtypeStruct((rows * out_len,), dtype),
        mesh=mesh,
        scratch_types=[pltpu.VMEM((ch,), dtype)] * _NBUF + [
            pltpu.SemaphoreType.DMA((_NBUF,)),
            pltpu.SemaphoreType.DMA((_NBUF,)),
        ],
    )
    def sc_copy(in_hbm, out_hbm, *rest):
        bufs, (rsem, wsem) = rest[:_NBUF], rest[_NBUF:]
        wid = lax.axis_index("s") * _NC + lax.axis_index("c")
        base_row = wid * rows_per_w

        def read_copy(t):
            row = base_row + t // nch
            off = pl.multiple_of(row * in_len + (t % nch) * ch, 8)
            slot = t % _NBUF
            return pltpu.make_async_copy(
                in_hbm.at[pl.ds(off, ch)], bufs[slot], rsem.at[slot]
            )

        def write_copy(t):
            row = base_row + t // nch
            off = pl.multiple_of(row * out_len + (t % nch) * ch, 8)
            slot = t % _NBUF
            return pltpu.make_async_copy(
                bufs[slot], out_hbm.at[pl.ds(off, ch)], wsem.at[slot]
            )

        for t in range(min(_AHEAD, ntask)):
            read_copy(t).start()
        for t in range(ntask):
            nt = t + _AHEAD
            if nt < ntask:
                if nt >= _NBUF:
                    write_copy(nt - _NBUF).wait()
                read_copy(nt).start()
            read_copy(t).wait()
            write_copy(t).start()
        for t in range(max(0, ntask - _NBUF), ntask):
            write_copy(t).wait()

    return sc_copy


def kernel(wav):
    s, b, c, length = wav.shape
    out_len = length - _SHIFT
    rows = s * b * c
    x = wav.reshape(rows * length)
    out = _make_sc_copy(rows, length, out_len, wav.dtype)(x)
    return out.reshape(s, b, c, out_len)


# speedup vs baseline: 6.9971x; 5.2326x over previous
import functools

import jax
import jax.numpy as jnp
from jax import lax
from jax.experimental import pallas as pl
from jax.experimental.pallas import tpu as pltpu
from jax.experimental.pallas import tpu_sc as plsc

_SHIFT = 8192
_LANE = 128

_NC = 2
_NS = 16
_NW = _NC * _NS

_NBUF = 3
_AHEAD = 2
_CHUNKS_PER_PAIR = 8


def _chunk_layout(out_len):
    total = out_len // _LANE
    n = _CHUNKS_PER_PAIR
    base, rem = divmod(total, n)
    lens = [base + (1 if i < rem else 0) for i in range(n)]
    offs, acc = [], 0
    for ln in lens:
        offs.append(acc)
        acc += ln
    return list(zip(offs, lens))


def _make_sc_copy(s, b, c, in_len, out_len, dtype):
    pairs = s * b
    pairs_per_w = pairs // _NW
    chunks = _chunk_layout(out_len)
    max_tiles = max(ln for _, ln in chunks)
    tasks = [(p, off, ln) for p in range(pairs_per_w) for off, ln in chunks]
    ntask = len(tasks)

    mesh = plsc.VectorSubcoreMesh(core_axis_name="c", subcore_axis_name="s")

    @functools.partial(
        pl.kernel,
        out_type=jax.ShapeDtypeStruct((s, b, c, out_len), dtype),
        mesh=mesh,
        scratch_types=[pltpu.VMEM((c, max_tiles * _LANE), dtype)] * _NBUF + [
            pltpu.SemaphoreType.DMA((_NBUF,)),
            pltpu.SemaphoreType.DMA((_NBUF,)),
        ],
    )
    def sc_copy(in_hbm, out_hbm, *rest):
        bufs, (rsem, wsem) = rest[:_NBUF], rest[_NBUF:]
        wid = lax.axis_index("s") * _NC + lax.axis_index("c")
        base_pair = wid * pairs_per_w

        def read_copy(t):
            p, off, ln = tasks[t]
            pair = base_pair + p
            si = pair // b
            bi = lax.rem(pair, b)
            slot = t % _NBUF
            return pltpu.make_async_copy(
                in_hbm.at[si, bi, :, pl.ds(off * _LANE, ln * _LANE)],
                bufs[slot].at[:, pl.ds(0, ln * _LANE)],
                rsem.at[slot],
            )

        def write_copy(t):
            p, off, ln = tasks[t]
            pair = base_pair + p
            si = pair // b
            bi = lax.rem(pair, b)
            slot = t % _NBUF
            return pltpu.make_async_copy(
                bufs[slot].at[:, pl.ds(0, ln * _LANE)],
                out_hbm.at[si, bi, :, pl.ds(off * _LANE, ln * _LANE)],
                wsem.at[slot],
            )

        for t in range(min(_AHEAD, ntask)):
            read_copy(t).start()
        for t in range(ntask):
            nt = t + _AHEAD
            if nt < ntask:
                if nt >= _NBUF:
                    write_copy(nt - _NBUF).wait()
                read_copy(nt).start()
            read_copy(t).wait()
            write_copy(t).start()
        for t in range(max(0, ntask - _NBUF), ntask):
            write_copy(t).wait()

    return sc_copy


def kernel(wav):
    s, b, c, length = wav.shape
    out_len = length - _SHIFT
    return _make_sc_copy(s, b, c, length, out_len, wav.dtype)(wav)


# speedup vs baseline: 7.0142x; 1.0024x over previous
import functools

import jax
import jax.numpy as jnp
from jax import lax
from jax.experimental import pallas as pl
from jax.experimental.pallas import tpu as pltpu
from jax.experimental.pallas import tpu_sc as plsc

_SHIFT = 8192
_LANE = 128

_NC = 2
_NS = 16
_NW = _NC * _NS

_NBUF = 3
_AHEAD = 2
_CHUNKS_PER_PAIR = 8


def _chunk_layout(out_len):
    total = out_len // _LANE
    n = _CHUNKS_PER_PAIR
    base, rem = divmod(total, n)
    lens = [base + (1 if i < rem else 0) for i in range(n)]
    offs, acc = [], 0
    for ln in lens:
        offs.append(acc)
        acc += ln
    return list(zip(offs, lens))


def _make_sc_copy(s, b, c, in_len, out_len, dtype):
    pairs = s * b
    pairs_per_w = pairs // _NW
    chunks = _chunk_layout(out_len)
    max_tiles = max(ln for _, ln in chunks)
    tasks = [(p, off, ln) for p in range(pairs_per_w) for off, ln in chunks]
    ntask = len(tasks)

    mesh = plsc.VectorSubcoreMesh(core_axis_name="c", subcore_axis_name="s")

    @functools.partial(
        pl.kernel,
        out_type=jax.ShapeDtypeStruct((s, b, c, out_len), dtype),
        mesh=mesh,
        scratch_types=[pltpu.VMEM((c, max_tiles * _LANE), dtype)] * _NBUF + [
            pltpu.SemaphoreType.DMA((_NBUF,)),
            pltpu.SemaphoreType.DMA((_NBUF,)),
        ],
        compiler_params=pltpu.CompilerParams(skip_device_barrier=True),
    )
    def sc_copy(in_hbm, out_hbm, *rest):
        bufs, (rsem, wsem) = rest[:_NBUF], rest[_NBUF:]
        wid = lax.axis_index("s") * _NC + lax.axis_index("c")
        base_pair = wid * pairs_per_w

        def read_copy(t):
            p, off, ln = tasks[t]
            pair = base_pair + p
            si = pair // b
            bi = lax.rem(pair, b)
            slot = t % _NBUF
            return pltpu.make_async_copy(
                in_hbm.at[si, bi, :, pl.ds(off * _LANE, ln * _LANE)],
                bufs[slot].at[:, pl.ds(0, ln * _LANE)],
                rsem.at[slot],
            )

        def write_copy(t):
            p, off, ln = tasks[t]
            pair = base_pair + p
            si = pair // b
            bi = lax.rem(pair, b)
            slot = t % _NBUF
            return pltpu.make_async_copy(
                bufs[slot].at[:, pl.ds(0, ln * _LANE)],
                out_hbm.at[si, bi, :, pl.ds(off * _LANE, ln * _LANE)],
                wsem.at[slot],
            )

        for t in range(min(_AHEAD, ntask)):
            read_copy(t).start()
        for t in range(ntask):
            nt = t + _AHEAD
            if nt < ntask:
                if nt >= _NBUF:
                    write_copy(nt - _NBUF).wait()
                read_copy(nt).start()
            read_copy(t).wait()
            write_copy(t).start()
        for t in range(max(0, ntask - _NBUF), ntask):
            write_copy(t).wait()

    return sc_copy


def kernel(wav):
    s, b, c, length = wav.shape
    out_len = length - _SHIFT
    return _make_sc_copy(s, b, c, length, out_len, wav.dtype)(wav)


# speedup vs baseline: 10.6013x; 1.5114x over previous
import jax
import jax.numpy as jnp
from jax.experimental import pallas as pl
from jax.experimental.pallas import tpu as pltpu

_SHIFT = 8192
_LANE = 128

_DEPTH = 16
_AHEAD = 8


def _make_tc_copy(s, b, c, in_len, out_len, dtype):
    total_tiles = out_len // _LANE
    half = total_tiles // 2
    chunks = [(0, half), (half, total_tiles - half)]
    tasks = [(si, bi, off, ln) for si in range(s) for bi in range(b)
             for off, ln in chunks]
    ntask = len(tasks)
    max_tiles = max(ln for _, ln in chunks)

    def body(in_ref, out_ref, *rest):
        bufs, (rsem, wsem) = rest[:_DEPTH], rest[_DEPTH:]

        def read_copy(t):
            si, bi, off, ln = tasks[t]
            slot = t % _DEPTH
            return pltpu.make_async_copy(
                in_ref.at[si, bi, :, pl.ds(off * _LANE, ln * _LANE)],
                bufs[slot].at[:, pl.ds(0, ln * _LANE)],
                rsem.at[slot],
            )

        def write_copy(t):
            si, bi, off, ln = tasks[t]
            slot = t % _DEPTH
            return pltpu.make_async_copy(
                bufs[slot].at[:, pl.ds(0, ln * _LANE)],
                out_ref.at[si, bi, :, pl.ds(off * _LANE, ln * _LANE)],
                wsem.at[slot],
            )

        for t in range(min(_AHEAD, ntask)):
            read_copy(t).start()
        for t in range(ntask):
            nt = t + _AHEAD
            if nt < ntask:
                if nt >= _DEPTH:
                    write_copy(nt - _DEPTH).wait()
                read_copy(nt).start()
            read_copy(t).wait()
            write_copy(t).start()
        for t in range(max(0, ntask - _DEPTH), ntask):
            write_copy(t).wait()

    return pl.pallas_call(
        body,
        in_specs=[pl.BlockSpec(memory_space=pl.ANY)],
        out_specs=pl.BlockSpec(memory_space=pl.ANY),
        out_shape=jax.ShapeDtypeStruct((s, b, c, out_len), dtype),
        scratch_shapes=[pltpu.VMEM((c, max_tiles * _LANE), dtype)] * _DEPTH + [
            pltpu.SemaphoreType.DMA((_DEPTH,)),
            pltpu.SemaphoreType.DMA((_DEPTH,)),
        ],
    )


def kernel(wav):
    s, b, c, length = wav.shape
    out_len = length - _SHIFT
    return _make_tc_copy(s, b, c, length, out_len, wav.dtype)(wav)
